# Initial kernel scaffold; baseline (speedup 1.0000x reference)
import jax
import jax.numpy as jnp
from jax.experimental import pallas as pl


def kernel(x, edge_index, W0, b0, W1, b1, W2, b2, Wc1, bc1, Wc2, bc2, Wc3, bc3):
    raise NotImplementedError("write your pallas kernel here")


# speedup vs baseline: 4.8924x; 4.8924x over previous
import functools

import jax
import jax.numpy as jnp
from jax import lax
from jax.experimental import pallas as pl
from jax.experimental.pallas import tpu as pltpu
from jax.experimental.pallas import tpu_sc as plsc

N = 10000
E = 320000
D = 128
H = 128
C = 18

NC = 2
NS = 16
NW = NC * NS
CHUNK = 128
NCH_W = 80
E_PAD = NW * NCH_W * CHUNK
N_PAD = 10240
SLAB = N_PAD // NS
TRASH = N
GRP = 16
NGRP = NCH_W // GRP

_sc_mesh = plsc.VectorSubcoreMesh(core_axis_name="c", subcore_axis_name="s",
                                  num_cores=NC, num_subcores=NS)


SPB = SLAB // CHUNK


def _fill_const(ref, nrow, width, val):
  def fr(r, _):
    for k in range(width // 16):
      ref[r, pl.ds(k * 16, 16)] = jnp.full((16,), val, jnp.float32)
    return 0
  lax.fori_loop(0, nrow, fr, 0)


@functools.partial(
    pl.kernel, mesh=_sc_mesh,
    out_type=jax.ShapeDtypeStruct((NC * N_PAD, D), jnp.float32),
    scratch_types=[
        pltpu.VMEM((GRP, CHUNK), jnp.int32),
        pltpu.VMEM((CHUNK, D), jnp.float32),
        pltpu.VMEM_SHARED((N_PAD, D), jnp.float32),
    ],
    name="sc_deg")
def _deg_kernel(dstm, deg_out, idx_d, ones_v, deg_sh):
  c = lax.axis_index("c")
  s = lax.axis_index("s")
  wid = s * NC + c
  rowbase = wid * NCH_W

  _fill_const(ones_v, CHUNK, D, 0.0)

  def zpiece(p, _):
    pltpu.sync_copy(ones_v, deg_sh.at[pl.ds(s * SLAB + p * CHUNK, CHUNK)])
    return 0

  lax.fori_loop(0, SPB, zpiece, 0)
  _fill_const(ones_v, CHUNK, D, 1.0)
  plsc.subcore_barrier()

  def group(g, _):
    pltpu.sync_copy(dstm.at[pl.ds(rowbase + g * GRP, GRP)], idx_d)

    def body(ci, _):
      pltpu.sync_copy(ones_v, deg_sh.at[idx_d.at[ci]], add=True)
      return 0

    lax.fori_loop(0, GRP, body, 0)
    return 0

  lax.fori_loop(0, NGRP, group, 0)
  plsc.subcore_barrier()

  def wpiece(p, _):
    rbase = s * SLAB + p * CHUNK
    pltpu.sync_copy(deg_sh.at[pl.ds(rbase, CHUNK)], ones_v)
    pltpu.sync_copy(ones_v, deg_out.at[pl.ds(c * N_PAD + rbase, CHUNK)])
    return 0

  lax.fori_loop(0, SPB, wpiece, 0)


def _make_agg_kernel(width):
  @functools.partial(
      pl.kernel, mesh=_sc_mesh,
      out_type=jax.ShapeDtypeStruct((NC * N_PAD, width), jnp.float32),
      scratch_types=[
          pltpu.VMEM((GRP, CHUNK), jnp.int32),
          pltpu.VMEM((GRP, CHUNK), jnp.int32),
          pltpu.VMEM((CHUNK, width), jnp.float32),
          pltpu.VMEM_SHARED((N_PAD, width), jnp.float32),
          pltpu.SemaphoreType.DMA,
      ],
      name=f"sc_agg{width}")
  def agg_kernel(h_hbm, srcm, dstm, agg_out, idx_s, idx_d, rows, agg_sh,
                 sem):
    c = lax.axis_index("c")
    s = lax.axis_index("s")
    wid = s * NC + c
    rowbase = wid * NCH_W

    _fill_const(rows, CHUNK, width, 0.0)

    def zpiece(p, _):
      pltpu.sync_copy(rows, agg_sh.at[pl.ds(s * SLAB + p * CHUNK, CHUNK)])
      return 0

    lax.fori_loop(0, SPB, zpiece, 0)
    plsc.subcore_barrier()

    def group(g, _):
      pltpu.sync_copy(srcm.at[pl.ds(rowbase + g * GRP, GRP)], idx_s)
      pltpu.sync_copy(dstm.at[pl.ds(rowbase + g * GRP, GRP)], idx_d)

      def body(ci, _):
        pltpu.async_copy(h_hbm.at[idx_s.at[ci]], rows, sem).wait()
        pltpu.sync_copy(rows, agg_sh.at[idx_d.at[ci]], add=True)
        return 0

      lax.fori_loop(0, GRP, body, 0)
      return 0

    lax.fori_loop(0, NGRP, group, 0)
    plsc.subcore_barrier()

    def wpiece(p, _):
      rbase = s * SLAB + p * CHUNK
      pltpu.sync_copy(agg_sh.at[pl.ds(rbase, CHUNK)], rows)
      pltpu.sync_copy(rows, agg_out.at[pl.ds(c * N_PAD + rbase, CHUNK)])
      return 0

    lax.fori_loop(0, SPB, wpiece, 0)

  return agg_kernel


_agg_kernel = _make_agg_kernel(D)


@functools.partial(
    pl.kernel, mesh=_sc_mesh,
    out_type=jax.ShapeDtypeStruct((E_PAD, H), jnp.float32),
    scratch_types=[
        pltpu.VMEM((NCH_W, CHUNK), jnp.int32),
        pltpu.VMEM((NCH_W, CHUNK), jnp.int32),
        pltpu.VMEM((CHUNK, H), jnp.float32),
        pltpu.VMEM((CHUNK, H), jnp.float32),
        pltpu.SemaphoreType.DMA,
        pltpu.SemaphoreType.DMA,
    ],
    name="sc_edge_emb")
def _edge_emb_kernel(t_hbm, u_hbm, srcm, dstm, z_out, idx_s, idx_d, rt, ru,
                     sem_t, sem_u):
  c = lax.axis_index("c")
  s = lax.axis_index("s")
  wid = s * NC + c
  rowbase = wid * NCH_W

  pltpu.sync_copy(srcm.at[pl.ds(rowbase, NCH_W)], idx_s)
  pltpu.sync_copy(dstm.at[pl.ds(rowbase, NCH_W)], idx_d)

  def body(ci, _):
    cp_t = pltpu.async_copy(t_hbm.at[idx_s.at[ci]], rt, sem_t)
    cp_u = pltpu.async_copy(u_hbm.at[idx_d.at[ci]], ru, sem_u)
    cp_t.wait()
    cp_u.wait()

    def add_row(r, _):
      for k in range(H // 16):
        sl = pl.ds(k * 16, 16)
        rt[r, sl] = rt[r, sl] + ru[r, sl]
      return 0

    lax.fori_loop(0, CHUNK, add_row, 0)
    pltpu.sync_copy(rt, z_out.at[pl.ds((rowbase + ci) * CHUNK, CHUNK)])
    return 0

  lax.fori_loop(0, NCH_W, body, 0)


def _node_tc_body(final, relu, refs):
  if final:
    aggp, degp, w, b, wt, wu, bc, t_out, u_out = refs
  else:
    aggp, degp, w, b, out = refs
  agg = aggp[0] + aggp[1]
  deg = jnp.maximum(degp[0][:, 0:1] + degp[1][:, 0:1], 1.0)
  h = lax.dot_general(agg / deg, w[...], (((1,), (0,)), ((), ())),
                      preferred_element_type=jnp.float32) + b[...]
  if relu:
    h = jnp.maximum(h, 0.0)
  if final:
    t_out[...] = lax.dot_general(h, wt[...], (((1,), (0,)), ((), ())),
                                 preferred_element_type=jnp.float32) + bc[...]
    u_out[...] = lax.dot_general(h, wu[...], (((1,), (0,)), ((), ())),
                                 preferred_element_type=jnp.float32)
  else:
    out[...] = h


_NODE_BLK = 1024


def _node_tc(aggp, degp, w, b, relu):
  grid = (N_PAD // _NODE_BLK,)
  return pl.pallas_call(
      lambda *refs: _node_tc_body(False, relu, refs),
      grid=grid,
      in_specs=[
          pl.BlockSpec((NC, _NODE_BLK, D), lambda i: (0, i, 0)),
          pl.BlockSpec((NC, _NODE_BLK, 16), lambda i: (0, i, 0)),
          pl.BlockSpec((D, H), lambda i: (0, 0)),
          pl.BlockSpec((1, H), lambda i: (0, 0)),
      ],
      out_specs=pl.BlockSpec((_NODE_BLK, H), lambda i: (i, 0)),
      out_shape=jax.ShapeDtypeStruct((N_PAD, H), jnp.float32),
  )(aggp, degp, w, b)


def _node_tc_final(aggp, degp, w, b, wt, wu, bc):
  grid = (N_PAD // _NODE_BLK,)
  return pl.pallas_call(
      lambda *refs: _node_tc_body(True, False, refs),
      grid=grid,
      in_specs=[
          pl.BlockSpec((NC, _NODE_BLK, D), lambda i: (0, i, 0)),
          pl.BlockSpec((NC, _NODE_BLK, 16), lambda i: (0, i, 0)),
          pl.BlockSpec((H, H), lambda i: (0, 0)),
          pl.BlockSpec((1, H), lambda i: (0, 0)),
          pl.BlockSpec((H, H), lambda i: (0, 0)),
          pl.BlockSpec((H, H), lambda i: (0, 0)),
          pl.BlockSpec((1, H), lambda i: (0, 0)),
      ],
      out_specs=[
          pl.BlockSpec((_NODE_BLK, H), lambda i: (i, 0)),
          pl.BlockSpec((_NODE_BLK, H), lambda i: (i, 0)),
      ],
      out_shape=[
          jax.ShapeDtypeStruct((N_PAD, H), jnp.float32),
          jax.ShapeDtypeStruct((N_PAD, H), jnp.float32),
      ],
  )(aggp, degp, w, b, wt, wu, bc)


_EDGE_BLK = 2000


def _edge_mlp_body(z_ref, w2, b2, w3, b3, out_ref):
  z = jnp.maximum(z_ref[...], 0.0)
  z2 = lax.dot_general(z, w2[...], (((1,), (0,)), ((), ())),
                       preferred_element_type=jnp.float32) + b2[...]
  z2 = jnp.maximum(z2, 0.0)
  out_ref[...] = lax.dot_general(z2, w3[...], (((1,), (0,)), ((), ())),
                                 preferred_element_type=jnp.float32) + b3[...]


def _edge_mlp(z1, w2, b2, w3, b3):
  grid = (E // _EDGE_BLK,)
  return pl.pallas_call(
      _edge_mlp_body,
      grid=grid,
      in_specs=[
          pl.BlockSpec((_EDGE_BLK, H), lambda i: (i, 0)),
          pl.BlockSpec((H, H // 2), lambda i: (0, 0)),
          pl.BlockSpec((1, H // 2), lambda i: (0, 0)),
          pl.BlockSpec((H // 2, C), lambda i: (0, 0)),
          pl.BlockSpec((1, C), lambda i: (0, 0)),
      ],
      out_specs=pl.BlockSpec((_EDGE_BLK, C), lambda i: (i, 0)),
      out_shape=jax.ShapeDtypeStruct((E, C), jnp.float32),
  )(z1, w2, b2, w3, b3)


def kernel(x, edge_index, W0, b0, W1, b1, W2, b2, Wc1, bc1, Wc2, bc2, Wc3,
           bc3):
  x_pad = jnp.zeros((N_PAD, D), jnp.float32).at[:N].set(x)
  pad_i = jnp.arange(E_PAD - E, dtype=jnp.int32)
  src = jnp.concatenate(
      [edge_index[0], pad_i % N]).reshape(-1, CHUNK)
  dst = jnp.concatenate(
      [edge_index[1], TRASH + pad_i % (N_PAD - N)]).reshape(-1, CHUNK)
  degp = _deg_kernel(dst).reshape(NC, N_PAD, D)[:, :, :16]
  aggp0 = _agg_kernel(x_pad, src, dst).reshape(NC, N_PAD, D)
  h = _node_tc(aggp0, degp, W0, b0.reshape(1, H), relu=True)
  aggp1 = _agg_kernel(h, src, dst).reshape(NC, N_PAD, D)
  h = _node_tc(aggp1, degp, W1, b1.reshape(1, H), relu=True)
  aggp2 = _agg_kernel(h, src, dst).reshape(NC, N_PAD, D)
  t_tab, u_tab = _node_tc_final(aggp2, degp, W2, b2.reshape(1, H),
                                Wc1[:H], Wc1[H:], bc1.reshape(1, H))
  z1 = _edge_emb_kernel(t_tab, u_tab, src, dst)
  return _edge_mlp(z1, Wc2, bc2.reshape(1, H // 2), Wc3,
                   bc3.reshape(1, C))
